# Initial kernel scaffold; baseline (speedup 1.0000x reference)
import jax
import jax.numpy as jnp
from jax.experimental import pallas as pl
from jax.experimental.pallas import tpu as pltpu
from jax.experimental.pallas import tpu_sc as plsc

_WINDOW = 128


def kernel(token_ids, weight):
    batch, seq = token_ids.shape
    num_indices = batch * seq
    embed = weight.shape[1]
    flat_ids = token_ids.reshape(1, num_indices).astype(jnp.int32)

    mesh = plsc.VectorSubcoreMesh(core_axis_name="core",
                                  subcore_axis_name="subcore")

    @pl.kernel(
        out_type=jax.ShapeDtypeStruct((num_indices, embed), weight.dtype),
        mesh=mesh,
    )
    def gather_kernel(w_hbm, i_hbm, o_hbm):
        def body(i_vmem, o_vmem):
            pltpu.sync_copy(w_hbm.at[i_vmem.at[0]], o_vmem)

        pltpu.emit_pipeline(
            body,
            grid=(num_indices // _WINDOW,),
            in_specs=[pl.BlockSpec((1, _WINDOW), index_map=lambda i: (0, i))],
            out_specs=[pl.BlockSpec((_WINDOW, embed),
                                    index_map=lambda i: (i, 0))],
            core_axis_name=("core", "subcore"),
            dimension_semantics=(pltpu.PARALLEL,),
        )(i_hbm, o_hbm)

    out = gather_kernel(weight, flat_ids)
    return out.reshape(batch, seq, embed)


# speedup vs baseline: 3.1779x; 3.1779x over previous
import jax
import jax.numpy as jnp
from jax import lax
from jax.experimental import pallas as pl
from jax.experimental.pallas import tpu as pltpu
from jax.experimental.pallas import tpu_sc as plsc

_NC = 2
_NS = 16
_CHUNK = 128


def kernel(token_ids, weight):
    batch, seq = token_ids.shape
    num_indices = batch * seq
    embed = weight.shape[1]
    nw = _NC * _NS
    per_w = num_indices // nw
    n_chunks = per_w // _CHUNK
    flat_ids = token_ids.reshape(num_indices).astype(jnp.int32)

    mesh = plsc.VectorSubcoreMesh(core_axis_name="c", subcore_axis_name="s")

    @pl.kernel(
        out_type=jax.ShapeDtypeStruct((num_indices, embed), weight.dtype),
        mesh=mesh,
        scratch_types=[
            pltpu.VMEM((_CHUNK,), jnp.int32),
            pltpu.VMEM((_CHUNK, embed), jnp.float32),
            pltpu.SemaphoreType.DMA,
        ],
        compiler_params=pltpu.CompilerParams(use_tc_tiling_on_sc=False),
    )
    def gather_kernel(w_hbm, i_hbm, o_hbm, idx_v, rows_v, sem):
        wid = lax.axis_index("s") * _NC + lax.axis_index("c")
        base = wid * per_w

        @pl.loop(0, n_chunks)
        def _(g):
            start = base + g * _CHUNK
            pltpu.sync_copy(i_hbm.at[pl.ds(start, _CHUNK)], idx_v)
            pltpu.async_copy(w_hbm.at[idx_v], rows_v, sem).wait()
            pltpu.sync_copy(rows_v, o_hbm.at[pl.ds(start, _CHUNK)])

    out = gather_kernel(weight, flat_ids)
    return out.reshape(batch, seq, embed)


# speedup vs baseline: 4.2342x; 1.3324x over previous
import jax
import jax.numpy as jnp
from jax import lax
from jax.experimental import pallas as pl
from jax.experimental.pallas import tpu as pltpu
from jax.experimental.pallas import tpu_sc as plsc

_NC = 2
_NS = 16
_GW = 128
_CHUNK = 512
_NBUF = 2


def kernel(token_ids, weight):
    batch, seq = token_ids.shape
    num_indices = batch * seq
    embed = weight.shape[1]
    nw = _NC * _NS
    per_w = num_indices // nw
    n_iter = per_w // _CHUNK
    flat_ids = token_ids.reshape(num_indices).astype(jnp.int32)

    mesh = plsc.VectorSubcoreMesh(core_axis_name="c", subcore_axis_name="s")

    @pl.kernel(
        out_type=jax.ShapeDtypeStruct((num_indices, embed), weight.dtype),
        mesh=mesh,
        scratch_types=[
            pltpu.VMEM((_CHUNK,), jnp.int32),
            pltpu.VMEM((_CHUNK,), jnp.int32),
            pltpu.VMEM((_CHUNK, embed), jnp.float32),
            pltpu.VMEM((_CHUNK, embed), jnp.float32),
            pltpu.SemaphoreType.DMA,
            pltpu.SemaphoreType.DMA,
            pltpu.SemaphoreType.DMA,
            pltpu.SemaphoreType.DMA,
            pltpu.SemaphoreType.DMA,
            pltpu.SemaphoreType.DMA,
        ],
        compiler_params=pltpu.CompilerParams(use_tc_tiling_on_sc=False),
    )
    def gather_kernel(w_hbm, i_hbm, o_hbm, idx0, idx1, r0, r1,
                      si0, si1, sg0, sg1, so0, so1):
        wid = lax.axis_index("s") * _NC + lax.axis_index("c")
        base = wid * per_w
        idxs = (idx0, idx1)
        rows = (r0, r1)
        sis = (si0, si1)
        sgs = (sg0, sg1)
        sos = (so0, so1)

        for b in range(_NBUF):
            pltpu.async_copy(
                i_hbm.at[pl.ds(base + b * _CHUNK, _CHUNK)], idxs[b], sis[b])

        @pl.loop(0, n_iter, step=_NBUF)
        def _(g):
            for b in range(_NBUF):
                k = g + b
                start = base + k * _CHUNK
                pltpu.make_async_copy(
                    i_hbm.at[pl.ds(start, _CHUNK)], idxs[b], sis[b]).wait()

                @pl.when(k >= _NBUF)
                def _():
                    pltpu.make_async_copy(
                        rows[b],
                        o_hbm.at[pl.ds(start - _NBUF * _CHUNK, _CHUNK)],
                        sos[b]).wait()

                for j in range(_CHUNK // _GW):
                    pltpu.async_copy(
                        w_hbm.at[idxs[b].at[pl.ds(j * _GW, _GW)]],
                        rows[b].at[pl.ds(j * _GW, _GW)],
                        sgs[b])
                for j in range(_CHUNK // _GW):
                    pltpu.make_async_copy(
                        w_hbm.at[idxs[b].at[pl.ds(j * _GW, _GW)]],
                        rows[b].at[pl.ds(j * _GW, _GW)],
                        sgs[b]).wait()

                @pl.when(k + _NBUF < n_iter)
                def _():
                    pltpu.async_copy(
                        i_hbm.at[pl.ds(start + _NBUF * _CHUNK, _CHUNK)],
                        idxs[b], sis[b])

                pltpu.async_copy(
                    rows[b], o_hbm.at[pl.ds(start, _CHUNK)], sos[b])

        for b in range(_NBUF):
            start = base + (n_iter - _NBUF + b) * _CHUNK
            pltpu.make_async_copy(
                rows[b], o_hbm.at[pl.ds(start, _CHUNK)], sos[b]).wait()

    out = gather_kernel(weight, flat_ids)
    return out.reshape(batch, seq, embed)


# speedup vs baseline: 4.2411x; 1.0016x over previous
import jax
import jax.numpy as jnp
from jax import lax
from jax.experimental import pallas as pl
from jax.experimental.pallas import tpu as pltpu
from jax.experimental.pallas import tpu_sc as plsc

_NC = 2
_NS = 16
_GW = 128
_CB = 4
_NBUF = 2


def kernel(token_ids, weight):
    batch, seq = token_ids.shape
    num_indices = batch * seq
    embed = weight.shape[1]
    nw = _NC * _NS
    batches_per_w = batch // nw
    n_iter = batches_per_w // _CB
    chunk = _CB * seq
    n_gw = -(-chunk // _GW)
    flat_ids = token_ids.reshape(num_indices).astype(jnp.int32)

    mesh = plsc.VectorSubcoreMesh(core_axis_name="c", subcore_axis_name="s")

    @pl.kernel(
        out_type=jax.ShapeDtypeStruct((batch, seq, embed), weight.dtype),
        mesh=mesh,
        scratch_types=[
            pltpu.VMEM((chunk,), jnp.int32),
            pltpu.VMEM((chunk,), jnp.int32),
            pltpu.VMEM((chunk, embed), jnp.float32),
            pltpu.VMEM((chunk, embed), jnp.float32),
            pltpu.SemaphoreType.DMA,
            pltpu.SemaphoreType.DMA,
            pltpu.SemaphoreType.DMA,
            pltpu.SemaphoreType.DMA,
            pltpu.SemaphoreType.DMA,
            pltpu.SemaphoreType.DMA,
        ],
        compiler_params=pltpu.CompilerParams(use_tc_tiling_on_sc=False),
    )
    def gather_kernel(w_hbm, i_hbm, o_hbm, idx0, idx1, r0, r1,
                      si0, si1, sg0, sg1, so0, so1):
        wid = lax.axis_index("s") * _NC + lax.axis_index("c")
        base_b = wid * batches_per_w
        idxs = (idx0, idx1)
        rows = (r0, r1)
        sis = (si0, si1)
        sgs = (sg0, sg1)
        sos = (so0, so1)

        def fire_writeback(b, b0):
            for bi in range(_CB):
                pltpu.async_copy(
                    rows[b].at[pl.ds(bi * seq, seq)],
                    o_hbm.at[b0 + bi], sos[b])

        def drain_writeback(b, b0):
            for bi in range(_CB):
                pltpu.make_async_copy(
                    rows[b].at[pl.ds(bi * seq, seq)],
                    o_hbm.at[b0 + bi], sos[b]).wait()

        for b in range(_NBUF):
            pltpu.async_copy(
                i_hbm.at[pl.ds((base_b + b * _CB) * seq, chunk)],
                idxs[b], sis[b])

        @pl.loop(0, n_iter, step=_NBUF)
        def _(g):
            for b in range(_NBUF):
                k = g + b
                b0 = base_b + k * _CB
                pltpu.make_async_copy(
                    i_hbm.at[pl.ds(b0 * seq, chunk)], idxs[b], sis[b]).wait()

                @pl.when(k >= _NBUF)
                def _():
                    drain_writeback(b, b0 - _NBUF * _CB)

                for j in range(n_gw):
                    w = min(_GW, chunk - j * _GW)
                    pltpu.async_copy(
                        w_hbm.at[idxs[b].at[pl.ds(j * _GW, w)]],
                        rows[b].at[pl.ds(j * _GW, w)],
                        sgs[b])
                for j in range(n_gw):
                    w = min(_GW, chunk - j * _GW)
                    pltpu.make_async_copy(
                        w_hbm.at[idxs[b].at[pl.ds(j * _GW, w)]],
                        rows[b].at[pl.ds(j * _GW, w)],
                        sgs[b]).wait()

                @pl.when(k + _NBUF < n_iter)
                def _():
                    pltpu.async_copy(
                        i_hbm.at[pl.ds((b0 + _NBUF * _CB) * seq, chunk)],
                        idxs[b], sis[b])

                fire_writeback(b, b0)

        for b in range(_NBUF):
            drain_writeback(b, base_b + (n_iter - _NBUF + b) * _CB)

    out = gather_kernel(weight, flat_ids)
    return out


# speedup vs baseline: 5.5498x; 1.3086x over previous
import jax
import jax.numpy as jnp
from jax import lax
from jax.experimental import pallas as pl
from jax.experimental.pallas import tpu as pltpu
from jax.experimental.pallas import tpu_sc as plsc

_NC = 2
_NS = 16
_GW = 128
_CHUNK = 256
_NBUF = 2
_PAD = 128


def kernel(token_ids, weight):
    batch, seq = token_ids.shape
    num_indices = batch * seq
    embed = weight.shape[1]
    nw = _NC * _NS
    per_w = num_indices // nw
    n_iter = per_w // _CHUNK
    flat_ids = token_ids.reshape(num_indices).astype(jnp.int32)
    wpad = jnp.pad(weight, ((0, 0), (0, _PAD - embed)))

    mesh = plsc.VectorSubcoreMesh(core_axis_name="c", subcore_axis_name="s")

    @pl.kernel(
        out_type=jax.ShapeDtypeStruct((num_indices, _PAD), weight.dtype),
        mesh=mesh,
        scratch_types=[
            pltpu.VMEM((_CHUNK,), jnp.int32),
            pltpu.VMEM((_CHUNK,), jnp.int32),
            pltpu.VMEM((_CHUNK, _PAD), jnp.float32),
            pltpu.VMEM((_CHUNK, _PAD), jnp.float32),
            pltpu.SemaphoreType.DMA,
            pltpu.SemaphoreType.DMA,
            pltpu.SemaphoreType.DMA,
            pltpu.SemaphoreType.DMA,
            pltpu.SemaphoreType.DMA,
            pltpu.SemaphoreType.DMA,
        ],
        compiler_params=pltpu.CompilerParams(use_tc_tiling_on_sc=True),
    )
    def gather_kernel(w_hbm, i_hbm, o_hbm, idx0, idx1, r0, r1,
                      si0, si1, sg0, sg1, so0, so1):
        wid = lax.axis_index("s") * _NC + lax.axis_index("c")
        base = wid * per_w
        idxs = (idx0, idx1)
        rows = (r0, r1)
        sis = (si0, si1)
        sgs = (sg0, sg1)
        sos = (so0, so1)

        for b in range(_NBUF):
            pltpu.async_copy(
                i_hbm.at[pl.ds(base + b * _CHUNK, _CHUNK)], idxs[b], sis[b])

        @pl.loop(0, n_iter, step=_NBUF)
        def _(g):
            for b in range(_NBUF):
                k = g + b
                start = base + k * _CHUNK
                pltpu.make_async_copy(
                    i_hbm.at[pl.ds(start, _CHUNK)], idxs[b], sis[b]).wait()

                @pl.when(k >= _NBUF)
                def _():
                    pltpu.make_async_copy(
                        rows[b],
                        o_hbm.at[pl.ds(start - _NBUF * _CHUNK, _CHUNK)],
                        sos[b]).wait()

                for j in range(_CHUNK // _GW):
                    pltpu.async_copy(
                        w_hbm.at[idxs[b].at[pl.ds(j * _GW, _GW)]],
                        rows[b].at[pl.ds(j * _GW, _GW)],
                        sgs[b])
                for j in range(_CHUNK // _GW):
                    pltpu.make_async_copy(
                        w_hbm.at[idxs[b].at[pl.ds(j * _GW, _GW)]],
                        rows[b].at[pl.ds(j * _GW, _GW)],
                        sgs[b]).wait()

                @pl.when(k + _NBUF < n_iter)
                def _():
                    pltpu.async_copy(
                        i_hbm.at[pl.ds(start + _NBUF * _CHUNK, _CHUNK)],
                        idxs[b], sis[b])

                pltpu.async_copy(
                    rows[b], o_hbm.at[pl.ds(start, _CHUNK)], sos[b])

        for b in range(_NBUF):
            start = base + (n_iter - _NBUF + b) * _CHUNK
            pltpu.make_async_copy(
                rows[b], o_hbm.at[pl.ds(start, _CHUNK)], sos[b]).wait()

    out = gather_kernel(wpad, flat_ids)
    return out[:, :embed].reshape(batch, seq, embed)


# speedup vs baseline: 5.5529x; 1.0006x over previous
import jax
import jax.numpy as jnp
from jax import lax
from jax.experimental import pallas as pl
from jax.experimental.pallas import tpu as pltpu
from jax.experimental.pallas import tpu_sc as plsc

_NC = 2
_NS = 16
_GW = 128
_CHUNK = 256
_NBUF = 2
_PAD = 128


def kernel(token_ids, weight):
    batch, seq = token_ids.shape
    num_indices = batch * seq
    embed = weight.shape[1]
    nw = _NC * _NS
    per_w = num_indices // nw
    n_iter = per_w // _CHUNK
    id_rows = per_w // _GW
    ids2 = token_ids.reshape(num_indices // _GW, _GW).astype(jnp.int32)
    wpad = jnp.pad(weight, ((0, 0), (0, _PAD - embed)))

    mesh = plsc.VectorSubcoreMesh(core_axis_name="c", subcore_axis_name="s")

    @pl.kernel(
        out_type=jax.ShapeDtypeStruct((num_indices, _PAD), weight.dtype),
        mesh=mesh,
        scratch_types=[
            pltpu.VMEM((id_rows, _GW), jnp.int32),
            pltpu.VMEM((_CHUNK, _PAD), jnp.float32),
            pltpu.VMEM((_CHUNK, _PAD), jnp.float32),
            pltpu.SemaphoreType.DMA,
            pltpu.SemaphoreType.DMA,
            pltpu.SemaphoreType.DMA,
            pltpu.SemaphoreType.DMA,
        ],
        compiler_params=pltpu.CompilerParams(use_tc_tiling_on_sc=True),
    )
    def gather_kernel(w_hbm, i_hbm, o_hbm, ids_v, r0, r1,
                      sg0, sg1, so0, so1):
        wid = lax.axis_index("s") * _NC + lax.axis_index("c")
        base = wid * per_w
        rows = (r0, r1)
        sgs = (sg0, sg1)
        sos = (so0, so1)

        pltpu.sync_copy(i_hbm.at[pl.ds(wid * id_rows, id_rows)], ids_v)

        @pl.loop(0, n_iter, step=_NBUF)
        def _(g):
            for b in range(_NBUF):
                k = g + b
                start = base + k * _CHUNK

                @pl.when(k >= _NBUF)
                def _():
                    pltpu.make_async_copy(
                        rows[b],
                        o_hbm.at[pl.ds(start - _NBUF * _CHUNK, _CHUNK)],
                        sos[b]).wait()

                for j in range(_CHUNK // _GW):
                    pltpu.async_copy(
                        w_hbm.at[ids_v.at[k * (_CHUNK // _GW) + j]],
                        rows[b].at[pl.ds(j * _GW, _GW)],
                        sgs[b])
                for j in range(_CHUNK // _GW):
                    pltpu.make_async_copy(
                        w_hbm.at[ids_v.at[k * (_CHUNK // _GW) + j]],
                        rows[b].at[pl.ds(j * _GW, _GW)],
                        sgs[b]).wait()

                pltpu.async_copy(
                    rows[b], o_hbm.at[pl.ds(start, _CHUNK)], sos[b])

        for b in range(_NBUF):
            start = base + (n_iter - _NBUF + b) * _CHUNK
            pltpu.make_async_copy(
                rows[b], o_hbm.at[pl.ds(start, _CHUNK)], sos[b]).wait()

    out = gather_kernel(wpad, ids2)
    return out[:, :embed].reshape(batch, seq, embed)


# speedup vs baseline: 5.5652x; 1.0022x over previous
import jax
import jax.numpy as jnp
from jax import lax
from jax.experimental import pallas as pl
from jax.experimental.pallas import tpu as pltpu
from jax.experimental.pallas import tpu_sc as plsc

_NC = 2
_NS = 16
_GW = 128
_CHUNK = 256
_PAD = 128


def kernel(token_ids, weight):
    batch, seq = token_ids.shape
    num_indices = batch * seq
    embed = weight.shape[1]
    nw = _NC * _NS
    per_w = num_indices // nw
    n_iter = per_w // _CHUNK
    id_rows = per_w // _GW
    gpc = _CHUNK // _GW
    ids2 = token_ids.reshape(num_indices // _GW, _GW).astype(jnp.int32)
    wpad = jnp.pad(weight, ((0, 0), (0, _PAD - embed)))

    mesh = plsc.VectorSubcoreMesh(core_axis_name="c", subcore_axis_name="s")

    @pl.kernel(
        out_type=jax.ShapeDtypeStruct((num_indices, _PAD), weight.dtype),
        mesh=mesh,
        scratch_types=[
            pltpu.VMEM((id_rows, _GW), jnp.int32),
            pltpu.VMEM((_CHUNK, _PAD), jnp.float32),
            pltpu.VMEM((_CHUNK, _PAD), jnp.float32),
            pltpu.SemaphoreType.DMA,
            pltpu.SemaphoreType.DMA,
            pltpu.SemaphoreType.DMA,
            pltpu.SemaphoreType.DMA,
        ],
        compiler_params=pltpu.CompilerParams(use_tc_tiling_on_sc=True),
    )
    def gather_kernel(w_hbm, i_hbm, o_hbm, ids_v, r0, r1,
                      sg0, sg1, so0, so1):
        wid = lax.axis_index("s") * _NC + lax.axis_index("c")
        base = wid * per_w
        rows = (r0, r1)
        sgs = (sg0, sg1)
        sos = (so0, so1)

        def fire_gathers(k, b):
            for j in range(gpc):
                pltpu.async_copy(
                    w_hbm.at[ids_v.at[k * gpc + j]],
                    rows[b].at[pl.ds(j * _GW, _GW)],
                    sgs[b])

        def drain_gathers(k, b):
            for j in range(gpc):
                pltpu.make_async_copy(
                    w_hbm.at[ids_v.at[k * gpc + j]],
                    rows[b].at[pl.ds(j * _GW, _GW)],
                    sgs[b]).wait()

        def fire_writeback(k, b):
            pltpu.async_copy(
                rows[b], o_hbm.at[pl.ds(base + k * _CHUNK, _CHUNK)], sos[b])

        def drain_writeback(k, b):
            pltpu.make_async_copy(
                rows[b], o_hbm.at[pl.ds(base + k * _CHUNK, _CHUNK)],
                sos[b]).wait()

        pltpu.sync_copy(i_hbm.at[pl.ds(wid * id_rows, id_rows)], ids_v)

        fire_gathers(0, 0)

        @pl.loop(0, n_iter, step=2)
        def _(g):
            for b in range(2):
                k = g + b
                nb = 1 - b

                @pl.when(k + 1 < n_iter)
                def _():
                    @pl.when(k >= 1)
                    def _():
                        drain_writeback(k - 1, nb)
                    fire_gathers(k + 1, nb)

                drain_gathers(k, b)
                fire_writeback(k, b)

        drain_writeback(n_iter - 1, (n_iter - 1) % 2)

    out = gather_kernel(wpad, ids2)
    return out[:, :embed].reshape(batch, seq, embed)
